# baseline (speedup 1.0000x reference)
import functools

import numpy as np
import jax
import jax.numpy as jnp
from jax import lax
from jax.experimental import pallas as pl
from jax.experimental.pallas import tpu as pltpu
from jax.experimental.pallas import tpu_sc as plsc

_MAX_LEN = 4096
_NC, _NS, _L = 2, 16, 16
_NW = _NC * _NS
_C = 8


def _pe_table(d_feature):
    pe = np.zeros((_MAX_LEN, d_feature), dtype=np.float32)
    position = np.arange(0, _MAX_LEN)[:, np.newaxis]
    scale_factor = -np.log(10000.0) / (d_feature // 2 - 1)
    div_term = np.exp(np.arange(0, d_feature // 2) * scale_factor)
    pe[:, :d_feature // 2] = np.sin(position * div_term)
    pe[:, d_feature // 2:2 * (d_feature // 2)] = np.cos(position * div_term)
    return jnp.asarray(pe)


def _sc_body(n_rows, d, x_hbm, pos_hbm, pe_hbm, out_hbm,
             idx_all, in0, in1, in2, in3, pe0, pe1, pe2, pe3,
             sem_in0, sem_in1, sem_in2, sem_in3,
             sem_pe0, sem_pe1, sem_pe2, sem_pe3,
             sem_out0, sem_out1, sem_out2, sem_out3):
    wid = lax.axis_index("s") * _NC + lax.axis_index("c")
    rows_per_w = n_rows // _NW
    base0 = wid * rows_per_w
    n_chunks = rows_per_w // _C
    n_vec = d // _L
    shift_r = n_vec.bit_length() - 1
    assert (1 << shift_r) == n_vec
    n_vec_chunk = _C * n_vec

    in_b = (in0, in1, in2, in3)
    pe_b = (pe0, pe1, pe2, pe3)
    sem_in = (sem_in0, sem_in1, sem_in2, sem_in3)
    sem_pe = (sem_pe0, sem_pe1, sem_pe2, sem_pe3)
    sem_out = (sem_out0, sem_out1, sem_out2, sem_out3)
    nbuf = 4

    pltpu.sync_copy(pos_hbm.at[pl.ds(base0, rows_per_w)], idx_all)

    def issue(g):
        b = g % nbuf
        base = base0 + g * _C
        ci = pltpu.async_copy(x_hbm.at[pl.ds(base, _C)], in_b[b], sem_in[b])
        cp = pltpu.async_copy(pe_hbm.at[idx_all.at[pl.ds(g * _C, _C)]],
                              pe_b[b], sem_pe[b])
        return ci, cp

    pending = [None] * nbuf
    out_desc = [None] * nbuf
    pending[0] = issue(0)
    pending[1] = issue(1)

    for g in range(n_chunks):
        b = g % nbuf
        ci, cp = pending[b]
        ci.wait()
        cp.wait()
        if g + 2 < n_chunks:
            nb = (g + 2) % nbuf
            if out_desc[nb] is not None:
                out_desc[nb].wait()
            pending[nb] = issue(g + 2)

        iv, pv = in_b[b], pe_b[b]

        @plsc.parallel_loop(0, n_vec_chunk, 1, unroll=4)
        def _(k):
            r = lax.shift_right_logical(k, shift_r)
            off = pl.multiple_of(lax.shift_left(k & (n_vec - 1), 4), _L)
            iv[r, pl.ds(off, _L)] = iv[r, pl.ds(off, _L)] + pv[r, pl.ds(off, _L)]

        out_desc[b] = pltpu.async_copy(
            iv, out_hbm.at[pl.ds(base0 + g * _C, _C)], sem_out[b])

    for b in range(nbuf):
        if out_desc[b] is not None:
            out_desc[b].wait()


def _make_sc_call(n_rows, d):
    mesh = plsc.VectorSubcoreMesh(
        core_axis_name="c", subcore_axis_name="s",
        num_cores=_NC, num_subcores=_NS)
    return pl.kernel(
        functools.partial(_sc_body, n_rows, d),
        out_type=jax.ShapeDtypeStruct((n_rows, d), jnp.float32),
        mesh=mesh,
        scratch_types=[
            pltpu.VMEM((n_rows // _NW,), jnp.int32),
        ] + [pltpu.VMEM((_C, d), jnp.float32)] * 8
          + [pltpu.SemaphoreType.DMA] * 12,
    )


def kernel(inputs, inputs_positions):
    b, t, d = inputs.shape
    n_rows = b * t
    x = inputs.reshape(n_rows, d)
    pos = inputs_positions.reshape(n_rows).astype(jnp.int32)
    pe = _pe_table(d)
    out = _make_sc_call(n_rows, d)(x, pos, pe)
    return out.reshape(b, t, d)


# speedup vs baseline: 1.0088x; 1.0088x over previous
import functools

import numpy as np
import jax
import jax.numpy as jnp
from jax import lax
from jax.experimental import pallas as pl
from jax.experimental.pallas import tpu as pltpu
from jax.experimental.pallas import tpu_sc as plsc

_MAX_LEN = 4096
_NC, _NS, _L = 2, 16, 16
_NW = _NC * _NS
_C = 8


def _pe_table(d_feature):
    pe = np.zeros((_MAX_LEN, d_feature), dtype=np.float32)
    position = np.arange(0, _MAX_LEN)[:, np.newaxis]
    scale_factor = -np.log(10000.0) / (d_feature // 2 - 1)
    div_term = np.exp(np.arange(0, d_feature // 2) * scale_factor)
    pe[:, :d_feature // 2] = np.sin(position * div_term)
    pe[:, d_feature // 2:2 * (d_feature // 2)] = np.cos(position * div_term)
    return jnp.asarray(pe)


def _sc_body(n_rows, d, x_hbm, pos_hbm, pe_hbm, out_hbm,
             idx_all, in0, in1, in2, in3, in4, in5,
             pe0, pe1, pe2, pe3, pe4, pe5,
             sem_in0, sem_in1, sem_in2, sem_in3, sem_in4, sem_in5,
             sem_pe0, sem_pe1, sem_pe2, sem_pe3, sem_pe4, sem_pe5,
             sem_out0, sem_out1, sem_out2, sem_out3, sem_out4, sem_out5):
    wid = lax.axis_index("s") * _NC + lax.axis_index("c")
    rows_per_w = n_rows // _NW
    base0 = wid * rows_per_w
    n_chunks = rows_per_w // _C
    n_vec = d // _L
    shift_r = n_vec.bit_length() - 1
    assert (1 << shift_r) == n_vec
    n_vec_chunk = _C * n_vec

    in_b = (in0, in1, in2, in3, in4, in5)
    pe_b = (pe0, pe1, pe2, pe3, pe4, pe5)
    sem_in = (sem_in0, sem_in1, sem_in2, sem_in3, sem_in4, sem_in5)
    sem_pe = (sem_pe0, sem_pe1, sem_pe2, sem_pe3, sem_pe4, sem_pe5)
    sem_out = (sem_out0, sem_out1, sem_out2, sem_out3, sem_out4, sem_out5)
    nbuf = 6
    ahead = 3

    pltpu.sync_copy(pos_hbm.at[pl.ds(base0, rows_per_w)], idx_all)

    def issue(g):
        b = g % nbuf
        base = base0 + g * _C
        ci = pltpu.async_copy(x_hbm.at[pl.ds(base, _C)], in_b[b], sem_in[b])
        cp = pltpu.async_copy(pe_hbm.at[idx_all.at[pl.ds(g * _C, _C)]],
                              pe_b[b], sem_pe[b])
        return ci, cp

    pending = [None] * nbuf
    out_desc = [None] * nbuf
    for g0 in range(ahead):
        pending[g0] = issue(g0)

    for g in range(n_chunks):
        b = g % nbuf
        ci, cp = pending[b]
        ci.wait()
        cp.wait()
        if g + ahead < n_chunks:
            nb = (g + ahead) % nbuf
            if out_desc[nb] is not None:
                out_desc[nb].wait()
            pending[nb] = issue(g + ahead)

        iv, pv = in_b[b], pe_b[b]

        @plsc.parallel_loop(0, n_vec_chunk, 1, unroll=8)
        def _(k):
            r = lax.shift_right_logical(k, shift_r)
            off = pl.multiple_of(lax.shift_left(k & (n_vec - 1), 4), _L)
            iv[r, pl.ds(off, _L)] = iv[r, pl.ds(off, _L)] + pv[r, pl.ds(off, _L)]

        out_desc[b] = pltpu.async_copy(
            iv, out_hbm.at[pl.ds(base0 + g * _C, _C)], sem_out[b])

    for b in range(nbuf):
        if out_desc[b] is not None:
            out_desc[b].wait()


def _make_sc_call(n_rows, d):
    mesh = plsc.VectorSubcoreMesh(
        core_axis_name="c", subcore_axis_name="s",
        num_cores=_NC, num_subcores=_NS)
    return pl.kernel(
        functools.partial(_sc_body, n_rows, d),
        out_type=jax.ShapeDtypeStruct((n_rows, d), jnp.float32),
        mesh=mesh,
        scratch_types=[
            pltpu.VMEM((n_rows // _NW,), jnp.int32),
        ] + [pltpu.VMEM((_C, d), jnp.float32)] * 12
          + [pltpu.SemaphoreType.DMA] * 18,
    )


def kernel(inputs, inputs_positions):
    b, t, d = inputs.shape
    n_rows = b * t
    x = inputs.reshape(n_rows, d)
    pos = inputs_positions.reshape(n_rows).astype(jnp.int32)
    pe = _pe_table(d)
    out = _make_sc_call(n_rows, d)(x, pos, pe)
    return out.reshape(b, t, d)


# speedup vs baseline: 1.0198x; 1.0110x over previous
import functools

import numpy as np
import jax
import jax.numpy as jnp
from jax import lax
from jax.experimental import pallas as pl
from jax.experimental.pallas import tpu as pltpu
from jax.experimental.pallas import tpu_sc as plsc

_MAX_LEN = 4096
_NC, _NS, _L = 2, 16, 16
_NW = _NC * _NS
_C = 16


def _pe_table(d_feature):
    pe = np.zeros((_MAX_LEN, d_feature), dtype=np.float32)
    position = np.arange(0, _MAX_LEN)[:, np.newaxis]
    scale_factor = -np.log(10000.0) / (d_feature // 2 - 1)
    div_term = np.exp(np.arange(0, d_feature // 2) * scale_factor)
    pe[:, :d_feature // 2] = np.sin(position * div_term)
    pe[:, d_feature // 2:2 * (d_feature // 2)] = np.cos(position * div_term)
    return jnp.asarray(pe)


def _sc_body(n_rows, d, x_hbm, pos_hbm, pe_hbm, out_hbm,
             idx_all, in0, in1, in2, pe0, pe1, pe2,
             sem_in0, sem_in1, sem_in2, sem_pe0, sem_pe1, sem_pe2,
             sem_out0, sem_out1, sem_out2):
    wid = lax.axis_index("s") * _NC + lax.axis_index("c")
    rows_per_w = n_rows // _NW
    base0 = wid * rows_per_w
    n_chunks = rows_per_w // _C
    n_vec = d // _L
    shift_r = n_vec.bit_length() - 1
    assert (1 << shift_r) == n_vec
    n_vec_chunk = _C * n_vec

    in_b = (in0, in1, in2)
    pe_b = (pe0, pe1, pe2)
    sem_in = (sem_in0, sem_in1, sem_in2)
    sem_pe = (sem_pe0, sem_pe1, sem_pe2)
    sem_out = (sem_out0, sem_out1, sem_out2)
    nbuf = 3

    pltpu.sync_copy(pos_hbm.at[pl.ds(base0, rows_per_w)], idx_all)

    def issue(g):
        b = g % nbuf
        base = base0 + g * _C
        ci = pltpu.async_copy(x_hbm.at[pl.ds(base, _C)], in_b[b], sem_in[b])
        cp = pltpu.async_copy(pe_hbm.at[idx_all.at[pl.ds(g * _C, _C)]],
                              pe_b[b], sem_pe[b])
        return ci, cp

    pending = [None] * nbuf
    out_desc = [None] * nbuf
    pending[0] = issue(0)
    pending[1] = issue(1)

    for g in range(n_chunks):
        b = g % nbuf
        ci, cp = pending[b]
        ci.wait()
        cp.wait()
        if g + 2 < n_chunks:
            nb = (g + 2) % nbuf
            if out_desc[nb] is not None:
                out_desc[nb].wait()
            pending[nb] = issue(g + 2)

        iv, pv = in_b[b], pe_b[b]

        @plsc.parallel_loop(0, n_vec_chunk, 1, unroll=8)
        def _(k):
            r = lax.shift_right_logical(k, shift_r)
            off = pl.multiple_of(lax.shift_left(k & (n_vec - 1), 4), _L)
            iv[r, pl.ds(off, _L)] = iv[r, pl.ds(off, _L)] + pv[r, pl.ds(off, _L)]

        out_desc[b] = pltpu.async_copy(
            iv, out_hbm.at[pl.ds(base0 + g * _C, _C)], sem_out[b])

    for b in range(nbuf):
        if out_desc[b] is not None:
            out_desc[b].wait()


def _make_sc_call(n_rows, d):
    mesh = plsc.VectorSubcoreMesh(
        core_axis_name="c", subcore_axis_name="s",
        num_cores=_NC, num_subcores=_NS)
    return pl.kernel(
        functools.partial(_sc_body, n_rows, d),
        out_type=jax.ShapeDtypeStruct((n_rows, d), jnp.float32),
        mesh=mesh,
        scratch_types=[
            pltpu.VMEM((n_rows // _NW,), jnp.int32),
        ] + [pltpu.VMEM((_C, d), jnp.float32)] * 6
          + [pltpu.SemaphoreType.DMA] * 9,
    )


def kernel(inputs, inputs_positions):
    b, t, d = inputs.shape
    n_rows = b * t
    x = inputs.reshape(n_rows, d)
    pos = inputs_positions.reshape(n_rows).astype(jnp.int32)
    pe = _pe_table(d)
    out = _make_sc_call(n_rows, d)(x, pos, pe)
    return out.reshape(b, t, d)
